# baseline (speedup 1.0000x reference)
import functools

import jax
import jax.numpy as jnp
from jax import lax
from jax.experimental import pallas as pl
from jax.experimental.pallas import tpu as pltpu
from jax.experimental.pallas import tpu_sc as plsc

B = 4096
NNUM = 13
NCAT = 26
V = 100000
D = 64

NC = 2
NS = 16
NW = NC * NS
CHUNK = B // NW


def _num_body(num_ref, w_ref, b_ref, out_ref):
    out_ref[...] = (num_ref[...][:, :, None] * w_ref[...][None, :, :]
                    + b_ref[...][None, :, :])


def _num_embed(numerical, num_W, num_bias):
    bb = 1024
    return pl.pallas_call(
        _num_body,
        grid=(B // bb,),
        in_specs=[
            pl.BlockSpec((bb, NNUM), lambda i: (i, 0)),
            pl.BlockSpec((NNUM, D), lambda i: (0, 0)),
            pl.BlockSpec((NNUM, D), lambda i: (0, 0)),
        ],
        out_specs=pl.BlockSpec((bb, NNUM, D), lambda i: (i, 0, 0)),
        out_shape=jax.ShapeDtypeStruct((B, NNUM, D), jnp.float32),
    )(numerical, num_W, num_bias)


def _sc_body(gidx_hbm, tab_hbm, bias_hbm, out_hbm, idx_v, rows_v, bias_v, gsem):
    wid = lax.axis_index("s") * NC + lax.axis_index("c")
    b0 = wid * CHUNK
    pltpu.sync_copy(bias_hbm, bias_v)

    def field_step(f, carry):
        pltpu.sync_copy(gidx_hbm.at[f, pl.ds(b0, CHUNK)], idx_v)
        pltpu.async_copy(tab_hbm.at[idx_v], rows_v, gsem).wait()
        bias_regs = [bias_v[f, pl.ds(16 * k, 16)] for k in range(D // 16)]

        def row_body(i, c):
            for k in range(D // 16):
                plsc.addupdate(rows_v.at[i, pl.ds(16 * k, 16)], bias_regs[k])
            return c

        lax.fori_loop(0, CHUNK, row_body, 0)
        pltpu.sync_copy(rows_v, out_hbm.at[pl.ds(b0, CHUNK), f])
        return carry

    lax.fori_loop(0, NCAT, field_step, 0)


def _cat_embed(gidx, tables_flat, cat_bias):
    mesh = plsc.VectorSubcoreMesh(core_axis_name="c", subcore_axis_name="s",
                                  num_cores=NC, num_subcores=NS)
    run = pl.kernel(
        _sc_body,
        out_type=jax.ShapeDtypeStruct((B, NCAT, D), jnp.float32),
        mesh=mesh,
        scratch_types=[
            pltpu.VMEM((CHUNK,), jnp.int32),
            pltpu.VMEM((CHUNK, D), jnp.float32),
            pltpu.VMEM((NCAT, D), jnp.float32),
            pltpu.SemaphoreType.DMA,
        ],
        compiler_params=pltpu.CompilerParams(use_tc_tiling_on_sc=False),
    )
    return run(gidx, tables_flat, cat_bias)


@jax.jit
def kernel(numerical, categorical, num_W, num_b, tables, num_token, cat_token,
           pos_enc):
    tables_flat = tables.reshape(NCAT * V, D)
    gidx = categorical.T + (jnp.arange(NCAT, dtype=jnp.int32) * V)[:, None]
    cat_bias = cat_token + pos_enc[NNUM:]
    num_bias = num_b + num_token + pos_enc[:NNUM]
    num_e = _num_embed(numerical, num_W, num_bias)
    cat_e = _cat_embed(gidx, tables_flat, cat_bias)
    return jnp.concatenate([num_e, cat_e], axis=1)
